# Initial kernel scaffold; baseline (speedup 1.0000x reference)
import jax
import jax.numpy as jnp
from jax.experimental import pallas as pl


def kernel(x, edge_index, W1, b1, g1, be1, W2, b2, g2, be2):
    raise NotImplementedError("write your pallas kernel here")


# speedup vs baseline: 12.6891x; 12.6891x over previous
import functools

import jax
import jax.numpy as jnp
from jax import lax
from jax.experimental import pallas as pl
from jax.experimental.pallas import tpu as pltpu
from jax.experimental.pallas import tpu_sc as plsc

_NC = 2
_NS = 16
_NW = _NC * _NS
_CH = 80


def _fill_rows(ref, nrows, ncols, value):
    vec = jnp.full((16,), value, jnp.float32)

    def body(i, c):
        for g in range(ncols // 16):
            ref[i, 16 * g:16 * (g + 1)] = vec
        return c

    lax.fori_loop(0, nrows, body, 0)


_DCH = 200


def _sweep_chunks(n, sid, fn):
    nck = n // _DCH
    npt = (nck + _NS - 1) // _NS

    def body(q, c):
        ck = sid + q * _NS

        @pl.when(ck < nck)
        def _():
            fn(ck * _DCH)

        return c

    lax.fori_loop(0, npt, body, 0)


def _sc_deg(dst1d, n):
    e = dst1d.shape[0]
    epw = e // _NW
    nit = epw // _CH
    mesh = plsc.VectorSubcoreMesh(core_axis_name="c", subcore_axis_name="s")

    @functools.partial(
        pl.kernel,
        out_type=jax.ShapeDtypeStruct((_NC, n, 16), jnp.float32),
        mesh=mesh,
        compiler_params=pltpu.CompilerParams(use_tc_tiling_on_sc=False),
        scratch_types=[
            pltpu.VMEM((_CH,), jnp.int32),
            pltpu.VMEM((_CH, 16), jnp.float32),
            pltpu.VMEM((_DCH, 16), jnp.float32),
            pltpu.VMEM_SHARED((n, 16), jnp.float32),
        ],
    )
    def k(dst_hbm, out_hbm, idx_v, ones_v, buf_v, acc_sh):
        cid = lax.axis_index("c")
        sid = lax.axis_index("s")
        wid = cid * _NS + sid
        _fill_rows(ones_v, _CH, 16, 1.0)
        _fill_rows(buf_v, _DCH, 16, 0.0)
        _sweep_chunks(n, sid,
                      lambda r0: pltpu.sync_copy(
                          buf_v, acc_sh.at[pl.ds(r0, _DCH), :]))
        plsc.subcore_barrier()

        def body(j, c):
            pltpu.sync_copy(dst_hbm.at[pl.ds(wid * epw + j * _CH, _CH)], idx_v)
            pltpu.sync_copy(ones_v, acc_sh.at[idx_v], add=True)
            return c

        lax.fori_loop(0, nit, body, 0)
        plsc.subcore_barrier()

        def drain(r0):
            pltpu.sync_copy(acc_sh.at[pl.ds(r0, _DCH), :], buf_v)
            pltpu.sync_copy(buf_v, out_hbm.at[cid, pl.ds(r0, _DCH), :])

        _sweep_chunks(n, sid, drain)

    return k(dst1d)


def _sc_agg(xs, src1d, dst1d):
    n, d = xs.shape
    e = src1d.shape[0]
    epw = e // _NW
    nit = epw // _CH
    mesh = plsc.VectorSubcoreMesh(core_axis_name="c", subcore_axis_name="s")

    @functools.partial(
        pl.kernel,
        out_type=jax.ShapeDtypeStruct((_NC, n, d), jnp.float32),
        mesh=mesh,
        scratch_types=[
            pltpu.VMEM((_CH,), jnp.int32),
            pltpu.VMEM((_CH,), jnp.int32),
            pltpu.VMEM((_CH, d), jnp.float32),
            pltpu.VMEM((_DCH, d), jnp.float32),
            pltpu.VMEM_SHARED((n, d), jnp.float32),
            pltpu.SemaphoreType.DMA,
        ],
    )
    def k(xs_hbm, src_hbm, dst_hbm, out_hbm, src_v, dst_v, rows_v, buf_v,
          acc_sh, sem):
        cid = lax.axis_index("c")
        sid = lax.axis_index("s")
        wid = cid * _NS + sid
        _fill_rows(buf_v, _DCH, d, 0.0)
        _sweep_chunks(n, sid,
                      lambda r0: pltpu.sync_copy(
                          buf_v, acc_sh.at[pl.ds(r0, _DCH), :]))
        plsc.subcore_barrier()

        def body(j, c):
            base = wid * epw + j * _CH
            pltpu.sync_copy(src_hbm.at[pl.ds(base, _CH)], src_v)
            pltpu.sync_copy(dst_hbm.at[pl.ds(base, _CH)], dst_v)
            pltpu.async_copy(xs_hbm.at[src_v], rows_v, sem).wait()
            pltpu.sync_copy(rows_v, acc_sh.at[dst_v], add=True)
            return c

        lax.fori_loop(0, nit, body, 0)
        plsc.subcore_barrier()

        def drain(r0):
            pltpu.sync_copy(acc_sh.at[pl.ds(r0, _DCH), :], buf_v)
            pltpu.sync_copy(buf_v, out_hbm.at[cid, pl.ds(r0, _DCH), :])

        _sweep_chunks(n, sid, drain)

    return k(xs, src1d, dst1d)


def _tc_prep(degp, x, bn):
    n, d = x.shape

    def body(degp_ref, x_ref, dinv_ref, xs_ref):
        deg = degp_ref[0][:, 0:1] + degp_ref[1][:, 0:1] + 1.0
        dinv = lax.rsqrt(jnp.maximum(deg, 1.0))
        dinv_ref[...] = dinv
        xs_ref[...] = x_ref[...] * dinv

    return pl.pallas_call(
        body,
        grid=(n // bn,),
        in_specs=[
            pl.BlockSpec((_NC, bn, 16), lambda i: (0, i, 0)),
            pl.BlockSpec((bn, d), lambda i: (i, 0)),
        ],
        out_specs=[
            pl.BlockSpec((bn, 1), lambda i: (i, 0)),
            pl.BlockSpec((bn, d), lambda i: (i, 0)),
        ],
        out_shape=[
            jax.ShapeDtypeStruct((n, 1), jnp.float32),
            jax.ShapeDtypeStruct((n, d), jnp.float32),
        ],
    )(degp, x)


def _tc_mmstats(sp, xin, dinv, w, b, bn):
    n, d = xin.shape
    nb = n // bn

    def body(sp_ref, x_ref, dinv_ref, w_ref, b_ref, h_ref, s1_ref, s2_ref):
        dv = dinv_ref[...]
        agg = dv * (sp_ref[0] + sp_ref[1]) + (dv * dv) * x_ref[...]
        h = jnp.dot(agg, w_ref[...], preferred_element_type=jnp.float32)
        h = h + b_ref[...]
        h_ref[...] = h
        s1_ref[...] = jnp.broadcast_to(jnp.sum(h, axis=0, keepdims=True)[None],
                                       (1, 8, h.shape[1]))
        s2_ref[...] = jnp.broadcast_to(
            jnp.sum(h * h, axis=0, keepdims=True)[None], (1, 8, h.shape[1]))

    return pl.pallas_call(
        body,
        grid=(nb,),
        in_specs=[
            pl.BlockSpec((_NC, bn, d), lambda i: (0, i, 0)),
            pl.BlockSpec((bn, d), lambda i: (i, 0)),
            pl.BlockSpec((bn, 1), lambda i: (i, 0)),
            pl.BlockSpec((d, d), lambda i: (0, 0)),
            pl.BlockSpec((1, d), lambda i: (0, 0)),
        ],
        out_specs=[
            pl.BlockSpec((bn, d), lambda i: (i, 0)),
            pl.BlockSpec((1, 8, d), lambda i: (i, 0, 0)),
            pl.BlockSpec((1, 8, d), lambda i: (i, 0, 0)),
        ],
        out_shape=[
            jax.ShapeDtypeStruct((n, d), jnp.float32),
            jax.ShapeDtypeStruct((nb, 8, d), jnp.float32),
            jax.ShapeDtypeStruct((nb, 8, d), jnp.float32),
        ],
    )(sp, xin, dinv, w, b)


def _bn_relu(h, s1, s2, g, be, n):
    mu = jnp.sum(s1, axis=(0, 1))[None] * (1.0 / (8.0 * n))
    ex2 = jnp.sum(s2, axis=(0, 1))[None] * (1.0 / (8.0 * n))
    rstd = lax.rsqrt(jnp.maximum(ex2 - mu * mu, 0.0) + 1e-5)
    return jnp.maximum((h - mu) * rstd * g + be, 0.0)


def _tc_bnrelu_mid(h, s1, s2, g, be, dinv, bn):
    n, d = h.shape
    nb = n // bn

    def body(h_ref, s1_ref, s2_ref, g_ref, be_ref, dinv_ref, h1_ref, xs_ref):
        h1 = _bn_relu(h_ref[...], s1_ref[...], s2_ref[...], g_ref[...],
                      be_ref[...], n)
        h1_ref[...] = h1
        xs_ref[...] = h1 * dinv_ref[...]

    return pl.pallas_call(
        body,
        grid=(nb,),
        in_specs=[
            pl.BlockSpec((bn, d), lambda i: (i, 0)),
            pl.BlockSpec((nb, 8, d), lambda i: (0, 0, 0)),
            pl.BlockSpec((nb, 8, d), lambda i: (0, 0, 0)),
            pl.BlockSpec((1, d), lambda i: (0, 0)),
            pl.BlockSpec((1, d), lambda i: (0, 0)),
            pl.BlockSpec((bn, 1), lambda i: (i, 0)),
        ],
        out_specs=[
            pl.BlockSpec((bn, d), lambda i: (i, 0)),
            pl.BlockSpec((bn, d), lambda i: (i, 0)),
        ],
        out_shape=[
            jax.ShapeDtypeStruct((n, d), jnp.float32),
            jax.ShapeDtypeStruct((n, d), jnp.float32),
        ],
    )(h, s1, s2, g, be, dinv)


def _tc_bnrelu_final(h, s1, s2, g, be, h1, bn):
    n, d = h.shape
    nb = n // bn

    def body(h_ref, s1_ref, s2_ref, g_ref, be_ref, h1_ref, out_ref):
        h2 = _bn_relu(h_ref[...], s1_ref[...], s2_ref[...], g_ref[...],
                      be_ref[...], n)
        out_ref[...] = h1_ref[...] + h2

    return pl.pallas_call(
        body,
        grid=(nb,),
        in_specs=[
            pl.BlockSpec((bn, d), lambda i: (i, 0)),
            pl.BlockSpec((nb, 8, d), lambda i: (0, 0, 0)),
            pl.BlockSpec((nb, 8, d), lambda i: (0, 0, 0)),
            pl.BlockSpec((1, d), lambda i: (0, 0)),
            pl.BlockSpec((1, d), lambda i: (0, 0)),
            pl.BlockSpec((bn, d), lambda i: (i, 0)),
        ],
        out_specs=pl.BlockSpec((bn, d), lambda i: (i, 0)),
        out_shape=jax.ShapeDtypeStruct((n, d), jnp.float32),
    )(h, s1, s2, g, be, h1)


def kernel(x, edge_index, W1, b1, g1, be1, W2, b2, g2, be2):
    n, d = x.shape
    e = edge_index.shape[1]
    assert e % (_NW * _CH) == 0 and n % _DCH == 0
    src1d = edge_index[0]
    dst1d = edge_index[1]
    b1r, g1r, be1r = b1.reshape(1, d), g1.reshape(1, d), be1.reshape(1, d)
    b2r, g2r, be2r = b2.reshape(1, d), g2.reshape(1, d), be2.reshape(1, d)
    bn = 1000

    degp = _sc_deg(dst1d, n)
    dinv, xs1 = _tc_prep(degp, x, bn)
    s1p = _sc_agg(xs1, src1d, dst1d)
    hpre1, a1, q1 = _tc_mmstats(s1p, x, dinv, W1, b1r, bn)
    h1, xs2 = _tc_bnrelu_mid(hpre1, a1, q1, g1r, be1r, dinv, bn)
    s2p = _sc_agg(xs2, src1d, dst1d)
    hpre2, a2, q2 = _tc_mmstats(s2p, h1, dinv, W2, b2r, bn)
    return _tc_bnrelu_final(hpre2, a2, q2, g2r, be2r, h1, bn)


# speedup vs baseline: 24.5644x; 1.9359x over previous
import functools

import jax
import jax.numpy as jnp
from jax import lax
from jax.experimental import pallas as pl
from jax.experimental.pallas import tpu as pltpu
from jax.experimental.pallas import tpu_sc as plsc

_NC = 2
_NS = 16
_NW = _NC * _NS
_CH = 80


def _fill_rows(ref, nrows, ncols, value):
    vec = jnp.full((16,), value, jnp.float32)

    def body(i, c):
        for g in range(ncols // 16):
            ref[i, 16 * g:16 * (g + 1)] = vec
        return c

    lax.fori_loop(0, nrows, body, 0)


def _sweep_chunks(n, dch, sid, fn):
    nck = n // dch
    npt = (nck + _NS - 1) // _NS

    def body(q, c):
        ck = sid + q * _NS

        @pl.when(ck < nck)
        def _():
            fn(ck * dch)

        return c

    lax.fori_loop(0, npt, body, 0)


def _sc_deg(dst1d, n):
    e = dst1d.shape[0]
    epw = e // _NW
    nit = epw // _CH
    mesh = plsc.VectorSubcoreMesh(core_axis_name="c", subcore_axis_name="s")

    @functools.partial(
        pl.kernel,
        out_type=jax.ShapeDtypeStruct((_NC, n, 16), jnp.float32),
        mesh=mesh,
        compiler_params=pltpu.CompilerParams(use_tc_tiling_on_sc=False),
        scratch_types=[
            pltpu.VMEM((e // _NW,), jnp.int32),
            pltpu.VMEM((_CH,), jnp.int32),
            pltpu.VMEM((_CH, 16), jnp.float32),
            pltpu.VMEM((200, 16), jnp.float32),
            pltpu.VMEM_SHARED((n, 16), jnp.float32),
        ],
    )
    def k(dst_hbm, out_hbm, flat_v, idx_v, ones_v, buf_v, acc_sh):
        cid = lax.axis_index("c")
        sid = lax.axis_index("s")
        wid = cid * _NS + sid
        _fill_rows(ones_v, _CH, 16, 1.0)
        _fill_rows(buf_v, 200, 16, 0.0)
        _sweep_chunks(n, 200, sid,
                      lambda r0: pltpu.sync_copy(
                          buf_v, acc_sh.at[pl.ds(r0, 200), :]))
        pltpu.sync_copy(dst_hbm.at[pl.ds(wid * epw, epw)], flat_v)
        plsc.subcore_barrier()

        def body(j, c):
            for c2 in range(_CH // 16):
                idx_v[16 * c2:16 * (c2 + 1)] = flat_v[pl.ds(j * _CH + 16 * c2,
                                                            16)]
            pltpu.sync_copy(ones_v, acc_sh.at[idx_v], add=True)
            return c

        lax.fori_loop(0, nit, body, 0)
        plsc.subcore_barrier()

        def drain(r0):
            pltpu.sync_copy(acc_sh.at[pl.ds(r0, 200), :], buf_v)
            pltpu.sync_copy(buf_v, out_hbm.at[cid, pl.ds(r0, 200), :])

        _sweep_chunks(n, 200, sid, drain)

    return k(dst1d)


def _sc_agg(xs, src1d, dst1d):
    n, d = xs.shape
    e = src1d.shape[0]
    epw = e // _NW
    nit = epw // _CH
    mesh = plsc.VectorSubcoreMesh(core_axis_name="c", subcore_axis_name="s")

    @functools.partial(
        pl.kernel,
        out_type=jax.ShapeDtypeStruct((_NC, n, d), jnp.float32),
        mesh=mesh,
        scratch_types=[
            pltpu.VMEM((_CH,), jnp.int32),
            pltpu.VMEM((_CH,), jnp.int32),
            pltpu.VMEM((_CH,), jnp.int32),
            pltpu.VMEM((_CH,), jnp.int32),
            pltpu.VMEM((_CH, d), jnp.float32),
            pltpu.VMEM((_CH, d), jnp.float32),
            pltpu.VMEM((_CH, d), jnp.float32),
            pltpu.VMEM_SHARED((n, d), jnp.float32),
            pltpu.SemaphoreType.DMA,
            pltpu.SemaphoreType.DMA,
            pltpu.SemaphoreType.DMA,
            pltpu.SemaphoreType.DMA,
        ],
    )
    def k(xs_hbm, src_hbm, dst_hbm, out_hbm, src0_v, dst0_v, src1_v, dst1_v,
          rows0_v, rows1_v, buf_v, acc_sh, semg0, semg1, semi0, semi1):
        cid = lax.axis_index("c")
        sid = lax.axis_index("s")
        wid = cid * _NS + sid
        _fill_rows(buf_v, _CH, d, 0.0)
        _sweep_chunks(n, _CH, sid,
                      lambda r0: pltpu.sync_copy(
                          buf_v, acc_sh.at[pl.ds(r0, _CH), :]))
        plsc.subcore_barrier()

        def load_idx(j, src_v, dst_v, semi):
            base = wid * epw + j * _CH
            pltpu.async_copy(src_hbm.at[pl.ds(base, _CH)], src_v, semi)
            pltpu.async_copy(dst_hbm.at[pl.ds(base, _CH)], dst_v, semi)

        def wait_idx(j, src_v, dst_v, semi):
            base = wid * epw + j * _CH
            pltpu.make_async_copy(src_hbm.at[pl.ds(base, _CH)], src_v,
                                  semi).wait()
            pltpu.make_async_copy(dst_hbm.at[pl.ds(base, _CH)], dst_v,
                                  semi).wait()

        def start_gather(src_v, rows, semg):
            pltpu.async_copy(xs_hbm.at[src_v], rows, semg)

        def finish(src_v, dst_v, rows, semg):
            pltpu.make_async_copy(xs_hbm.at[src_v], rows, semg).wait()
            pltpu.sync_copy(rows, acc_sh.at[dst_v], add=True)

        load_idx(0, src0_v, dst0_v, semi0)
        wait_idx(0, src0_v, dst0_v, semi0)
        start_gather(src0_v, rows0_v, semg0)
        load_idx(1, src1_v, dst1_v, semi1)

        def body(q, c):
            j0 = 2 * q
            wait_idx(j0 + 1, src1_v, dst1_v, semi1)
            start_gather(src1_v, rows1_v, semg1)
            finish(src0_v, dst0_v, rows0_v, semg0)

            @pl.when(j0 + 2 < nit)
            def _():
                load_idx(j0 + 2, src0_v, dst0_v, semi0)
                wait_idx(j0 + 2, src0_v, dst0_v, semi0)
                start_gather(src0_v, rows0_v, semg0)

            finish(src1_v, dst1_v, rows1_v, semg1)

            @pl.when(j0 + 3 < nit)
            def _():
                load_idx(j0 + 3, src1_v, dst1_v, semi1)

            return c

        lax.fori_loop(0, nit // 2, body, 0)
        if nit % 2:
            finish(src0_v, dst0_v, rows0_v, semg0)
        plsc.subcore_barrier()

        def drain(r0):
            pltpu.sync_copy(acc_sh.at[pl.ds(r0, _CH), :], buf_v)
            pltpu.sync_copy(buf_v, out_hbm.at[cid, pl.ds(r0, _CH), :])

        _sweep_chunks(n, _CH, sid, drain)

    return k(xs, src1d, dst1d)


def _tc_prep(degp, x, bn):
    n, d = x.shape

    def body(degp_ref, x_ref, dinv_ref, xs_ref):
        deg = degp_ref[0][:, 0:1] + degp_ref[1][:, 0:1] + 1.0
        dinv = lax.rsqrt(jnp.maximum(deg, 1.0))
        dinv_ref[...] = dinv
        xs_ref[...] = x_ref[...] * dinv

    return pl.pallas_call(
        body,
        grid=(n // bn,),
        in_specs=[
            pl.BlockSpec((_NC, bn, 16), lambda i: (0, i, 0)),
            pl.BlockSpec((bn, d), lambda i: (i, 0)),
        ],
        out_specs=[
            pl.BlockSpec((bn, 1), lambda i: (i, 0)),
            pl.BlockSpec((bn, d), lambda i: (i, 0)),
        ],
        out_shape=[
            jax.ShapeDtypeStruct((n, 1), jnp.float32),
            jax.ShapeDtypeStruct((n, d), jnp.float32),
        ],
    )(degp, x)


def _tc_mmstats(sp, xin, dinv, w, b, bn):
    n, d = xin.shape
    nb = n // bn

    def body(sp_ref, x_ref, dinv_ref, w_ref, b_ref, h_ref, s1_ref, s2_ref):
        dv = dinv_ref[...]
        agg = dv * (sp_ref[0] + sp_ref[1]) + (dv * dv) * x_ref[...]
        h = jnp.dot(agg, w_ref[...], preferred_element_type=jnp.float32)
        h = h + b_ref[...]
        h_ref[...] = h
        s1_ref[...] = jnp.broadcast_to(jnp.sum(h, axis=0, keepdims=True)[None],
                                       (1, 8, h.shape[1]))
        s2_ref[...] = jnp.broadcast_to(
            jnp.sum(h * h, axis=0, keepdims=True)[None], (1, 8, h.shape[1]))

    return pl.pallas_call(
        body,
        grid=(nb,),
        in_specs=[
            pl.BlockSpec((_NC, bn, d), lambda i: (0, i, 0)),
            pl.BlockSpec((bn, d), lambda i: (i, 0)),
            pl.BlockSpec((bn, 1), lambda i: (i, 0)),
            pl.BlockSpec((d, d), lambda i: (0, 0)),
            pl.BlockSpec((1, d), lambda i: (0, 0)),
        ],
        out_specs=[
            pl.BlockSpec((bn, d), lambda i: (i, 0)),
            pl.BlockSpec((1, 8, d), lambda i: (i, 0, 0)),
            pl.BlockSpec((1, 8, d), lambda i: (i, 0, 0)),
        ],
        out_shape=[
            jax.ShapeDtypeStruct((n, d), jnp.float32),
            jax.ShapeDtypeStruct((nb, 8, d), jnp.float32),
            jax.ShapeDtypeStruct((nb, 8, d), jnp.float32),
        ],
    )(sp, xin, dinv, w, b)


def _bn_relu(h, s1, s2, g, be, n):
    mu = jnp.sum(s1, axis=(0, 1))[None] * (1.0 / (8.0 * n))
    ex2 = jnp.sum(s2, axis=(0, 1))[None] * (1.0 / (8.0 * n))
    rstd = lax.rsqrt(jnp.maximum(ex2 - mu * mu, 0.0) + 1e-5)
    return jnp.maximum((h - mu) * rstd * g + be, 0.0)


def _tc_bnrelu_mid(h, s1, s2, g, be, dinv, bn):
    n, d = h.shape
    nb = n // bn

    def body(h_ref, s1_ref, s2_ref, g_ref, be_ref, dinv_ref, h1_ref, xs_ref):
        h1 = _bn_relu(h_ref[...], s1_ref[...], s2_ref[...], g_ref[...],
                      be_ref[...], n)
        h1_ref[...] = h1
        xs_ref[...] = h1 * dinv_ref[...]

    return pl.pallas_call(
        body,
        grid=(nb,),
        in_specs=[
            pl.BlockSpec((bn, d), lambda i: (i, 0)),
            pl.BlockSpec((nb, 8, d), lambda i: (0, 0, 0)),
            pl.BlockSpec((nb, 8, d), lambda i: (0, 0, 0)),
            pl.BlockSpec((1, d), lambda i: (0, 0)),
            pl.BlockSpec((1, d), lambda i: (0, 0)),
            pl.BlockSpec((bn, 1), lambda i: (i, 0)),
        ],
        out_specs=[
            pl.BlockSpec((bn, d), lambda i: (i, 0)),
            pl.BlockSpec((bn, d), lambda i: (i, 0)),
        ],
        out_shape=[
            jax.ShapeDtypeStruct((n, d), jnp.float32),
            jax.ShapeDtypeStruct((n, d), jnp.float32),
        ],
    )(h, s1, s2, g, be, dinv)


def _tc_bnrelu_final(h, s1, s2, g, be, h1, bn):
    n, d = h.shape
    nb = n // bn

    def body(h_ref, s1_ref, s2_ref, g_ref, be_ref, h1_ref, out_ref):
        h2 = _bn_relu(h_ref[...], s1_ref[...], s2_ref[...], g_ref[...],
                      be_ref[...], n)
        out_ref[...] = h1_ref[...] + h2

    return pl.pallas_call(
        body,
        grid=(nb,),
        in_specs=[
            pl.BlockSpec((bn, d), lambda i: (i, 0)),
            pl.BlockSpec((nb, 8, d), lambda i: (0, 0, 0)),
            pl.BlockSpec((nb, 8, d), lambda i: (0, 0, 0)),
            pl.BlockSpec((1, d), lambda i: (0, 0)),
            pl.BlockSpec((1, d), lambda i: (0, 0)),
            pl.BlockSpec((bn, d), lambda i: (i, 0)),
        ],
        out_specs=pl.BlockSpec((bn, d), lambda i: (i, 0)),
        out_shape=jax.ShapeDtypeStruct((n, d), jnp.float32),
    )(h, s1, s2, g, be, h1)


def kernel(x, edge_index, W1, b1, g1, be1, W2, b2, g2, be2):
    n, d = x.shape
    e = edge_index.shape[1]
    assert e % (_NW * _CH) == 0 and n % 200 == 0 and n % _CH == 0
    src1d = edge_index[0]
    dst1d = edge_index[1]
    b1r, g1r, be1r = b1.reshape(1, d), g1.reshape(1, d), be1.reshape(1, d)
    b2r, g2r, be2r = b2.reshape(1, d), g2.reshape(1, d), be2.reshape(1, d)
    bn = 1000

    degp = _sc_deg(dst1d, n)
    dinv, xs1 = _tc_prep(degp, x, bn)
    s1p = _sc_agg(xs1, src1d, dst1d)
    hpre1, a1, q1 = _tc_mmstats(s1p, x, dinv, W1, b1r, bn)
    h1, xs2 = _tc_bnrelu_mid(hpre1, a1, q1, g1r, be1r, dinv, bn)
    s2p = _sc_agg(xs2, src1d, dst1d)
    hpre2, a2, q2 = _tc_mmstats(s2p, h1, dinv, W2, b2r, bn)
    return _tc_bnrelu_final(hpre2, a2, q2, g2r, be2r, h1, bn)


# speedup vs baseline: 28.2517x; 1.1501x over previous
import functools

import jax
import jax.numpy as jnp
from jax import lax
from jax.experimental import pallas as pl
from jax.experimental.pallas import tpu as pltpu
from jax.experimental.pallas import tpu_sc as plsc

_NC = 2
_NS = 16
_NW = _NC * _NS
_CH = 80


def _fill_rows(ref, nrows, ncols, value):
    vec = jnp.full((16,), value, jnp.float32)

    def body(i, c):
        for g in range(ncols // 16):
            ref[i, 16 * g:16 * (g + 1)] = vec
        return c

    lax.fori_loop(0, nrows, body, 0)


def _sweep_chunks(n, dch, sid, fn):
    nck = n // dch
    npt = (nck + _NS - 1) // _NS

    def body(q, c):
        ck = sid + q * _NS

        @pl.when(ck < nck)
        def _():
            fn(ck * dch)

        return c

    lax.fori_loop(0, npt, body, 0)


def _sc_deg(dst1d, n):
    e = dst1d.shape[0]
    epw = e // _NW
    nit = epw // _CH
    mesh = plsc.VectorSubcoreMesh(core_axis_name="c", subcore_axis_name="s")

    @functools.partial(
        pl.kernel,
        out_type=jax.ShapeDtypeStruct((_NC, n, 16), jnp.float32),
        mesh=mesh,
        compiler_params=pltpu.CompilerParams(use_tc_tiling_on_sc=False),
        scratch_types=[
            pltpu.VMEM((e // _NW,), jnp.int32),
            pltpu.VMEM((_CH,), jnp.int32),
            pltpu.VMEM((_CH, 16), jnp.float32),
            pltpu.VMEM((200, 16), jnp.float32),
            pltpu.VMEM_SHARED((n, 16), jnp.float32),
        ],
    )
    def k(dst_hbm, out_hbm, flat_v, idx_v, ones_v, buf_v, acc_sh):
        cid = lax.axis_index("c")
        sid = lax.axis_index("s")
        wid = cid * _NS + sid
        _fill_rows(ones_v, _CH, 16, 1.0)
        _fill_rows(buf_v, 200, 16, 0.0)
        _sweep_chunks(n, 200, sid,
                      lambda r0: pltpu.sync_copy(
                          buf_v, acc_sh.at[pl.ds(r0, 200), :]))
        pltpu.sync_copy(dst_hbm.at[pl.ds(wid * epw, epw)], flat_v)
        plsc.subcore_barrier()

        def body(j, c):
            for c2 in range(_CH // 16):
                idx_v[16 * c2:16 * (c2 + 1)] = flat_v[pl.ds(j * _CH + 16 * c2,
                                                            16)]
            pltpu.sync_copy(ones_v, acc_sh.at[idx_v], add=True)
            return c

        lax.fori_loop(0, nit, body, 0)
        plsc.subcore_barrier()

        def drain(r0):
            pltpu.sync_copy(acc_sh.at[pl.ds(r0, 200), :], buf_v)
            pltpu.sync_copy(buf_v, out_hbm.at[cid, pl.ds(r0, 200), :])

        _sweep_chunks(n, 200, sid, drain)

    return k(dst1d)


def _sc_agg(xs, src1d, dst1d):
    n, d = xs.shape
    e = src1d.shape[0]
    epw = e // _NW
    nit = epw // _CH
    mesh = plsc.VectorSubcoreMesh(core_axis_name="c", subcore_axis_name="s")

    @functools.partial(
        pl.kernel,
        out_type=jax.ShapeDtypeStruct((_NC, n, d), jnp.float32),
        mesh=mesh,
        scratch_types=[
            pltpu.VMEM((e // _NW,), jnp.int32),
            pltpu.VMEM((_CH,), jnp.int32),
            pltpu.VMEM((_CH,), jnp.int32),
            pltpu.VMEM((_CH, d), jnp.float32),
            pltpu.VMEM((_CH, d), jnp.float32),
            pltpu.VMEM((_CH, d), jnp.float32),
            pltpu.VMEM_SHARED((n, d), jnp.float32),
            pltpu.SemaphoreType.DMA,
            pltpu.SemaphoreType.DMA,
            pltpu.SemaphoreType.DMA,
            pltpu.SemaphoreType.DMA,
        ],
    )
    def k(xs_hbm, src_hbm, dst_hbm, out_hbm, srcf_v, dst0_v, dst1_v,
          rows0_v, rows1_v, buf_v, acc_sh, semg0, semg1, semi0, semi1):
        cid = lax.axis_index("c")
        sid = lax.axis_index("s")
        wid = cid * _NS + sid
        _fill_rows(buf_v, _CH, d, 0.0)
        _sweep_chunks(n, _CH, sid,
                      lambda r0: pltpu.sync_copy(
                          buf_v, acc_sh.at[pl.ds(r0, _CH), :]))
        pltpu.sync_copy(src_hbm.at[pl.ds(wid * epw, epw)], srcf_v)
        plsc.subcore_barrier()

        def load_dst(j, dst_v, semi):
            pltpu.async_copy(dst_hbm.at[pl.ds(wid * epw + j * _CH, _CH)],
                             dst_v, semi)

        def start_gather(j, rows, semg):
            pltpu.async_copy(xs_hbm.at[srcf_v.at[pl.ds(j * _CH, _CH)]],
                             rows, semg)

        def finish(j, dst_v, semi, rows, semg):
            pltpu.make_async_copy(xs_hbm.at[srcf_v.at[pl.ds(j * _CH, _CH)]],
                                  rows, semg).wait()
            pltpu.make_async_copy(dst_hbm.at[pl.ds(wid * epw + j * _CH, _CH)],
                                  dst_v, semi).wait()
            pltpu.sync_copy(rows, acc_sh.at[dst_v], add=True)

        load_dst(0, dst0_v, semi0)
        start_gather(0, rows0_v, semg0)
        load_dst(1, dst1_v, semi1)
        start_gather(1, rows1_v, semg1)

        def body(q, c):
            j0 = 2 * q
            finish(j0, dst0_v, semi0, rows0_v, semg0)

            @pl.when(j0 + 2 < nit)
            def _():
                load_dst(j0 + 2, dst0_v, semi0)
                start_gather(j0 + 2, rows0_v, semg0)

            finish(j0 + 1, dst1_v, semi1, rows1_v, semg1)

            @pl.when(j0 + 3 < nit)
            def _():
                load_dst(j0 + 3, dst1_v, semi1)
                start_gather(j0 + 3, rows1_v, semg1)

            return c

        lax.fori_loop(0, nit // 2, body, 0)
        if nit % 2:
            finish(nit - 1, dst0_v, semi0, rows0_v, semg0)
        plsc.subcore_barrier()

        def drain(r0):
            pltpu.sync_copy(acc_sh.at[pl.ds(r0, _CH), :], buf_v)
            pltpu.sync_copy(buf_v, out_hbm.at[cid, pl.ds(r0, _CH), :])

        _sweep_chunks(n, _CH, sid, drain)

    return k(xs, src1d, dst1d)


def _tc_prep(degp, x, bn):
    n, d = x.shape

    def body(degp_ref, x_ref, dinv_ref, xs_ref):
        deg = degp_ref[0][:, 0:1] + degp_ref[1][:, 0:1] + 1.0
        dinv = lax.rsqrt(jnp.maximum(deg, 1.0))
        dinv_ref[...] = dinv
        xs_ref[...] = x_ref[...] * dinv

    return pl.pallas_call(
        body,
        grid=(n // bn,),
        in_specs=[
            pl.BlockSpec((_NC, bn, 16), lambda i: (0, i, 0)),
            pl.BlockSpec((bn, d), lambda i: (i, 0)),
        ],
        out_specs=[
            pl.BlockSpec((bn, 1), lambda i: (i, 0)),
            pl.BlockSpec((bn, d), lambda i: (i, 0)),
        ],
        out_shape=[
            jax.ShapeDtypeStruct((n, 1), jnp.float32),
            jax.ShapeDtypeStruct((n, d), jnp.float32),
        ],
    )(degp, x)


def _tc_mmstats(sp, xin, dinv, w, b, bn):
    n, d = xin.shape
    nb = n // bn

    def body(sp_ref, x_ref, dinv_ref, w_ref, b_ref, h_ref, s1_ref, s2_ref):
        dv = dinv_ref[...]
        agg = dv * (sp_ref[0] + sp_ref[1]) + (dv * dv) * x_ref[...]
        h = jnp.dot(agg, w_ref[...], preferred_element_type=jnp.float32)
        h = h + b_ref[...]
        h_ref[...] = h
        s1_ref[...] = jnp.broadcast_to(jnp.sum(h, axis=0, keepdims=True)[None],
                                       (1, 8, h.shape[1]))
        s2_ref[...] = jnp.broadcast_to(
            jnp.sum(h * h, axis=0, keepdims=True)[None], (1, 8, h.shape[1]))

    return pl.pallas_call(
        body,
        grid=(nb,),
        in_specs=[
            pl.BlockSpec((_NC, bn, d), lambda i: (0, i, 0)),
            pl.BlockSpec((bn, d), lambda i: (i, 0)),
            pl.BlockSpec((bn, 1), lambda i: (i, 0)),
            pl.BlockSpec((d, d), lambda i: (0, 0)),
            pl.BlockSpec((1, d), lambda i: (0, 0)),
        ],
        out_specs=[
            pl.BlockSpec((bn, d), lambda i: (i, 0)),
            pl.BlockSpec((1, 8, d), lambda i: (i, 0, 0)),
            pl.BlockSpec((1, 8, d), lambda i: (i, 0, 0)),
        ],
        out_shape=[
            jax.ShapeDtypeStruct((n, d), jnp.float32),
            jax.ShapeDtypeStruct((nb, 8, d), jnp.float32),
            jax.ShapeDtypeStruct((nb, 8, d), jnp.float32),
        ],
    )(sp, xin, dinv, w, b)


def _bn_relu(h, s1, s2, g, be, n):
    mu = jnp.sum(s1, axis=(0, 1))[None] * (1.0 / (8.0 * n))
    ex2 = jnp.sum(s2, axis=(0, 1))[None] * (1.0 / (8.0 * n))
    rstd = lax.rsqrt(jnp.maximum(ex2 - mu * mu, 0.0) + 1e-5)
    return jnp.maximum((h - mu) * rstd * g + be, 0.0)


def _tc_bnrelu_mid(h, s1, s2, g, be, dinv, bn):
    n, d = h.shape
    nb = n // bn

    def body(h_ref, s1_ref, s2_ref, g_ref, be_ref, dinv_ref, h1_ref, xs_ref):
        h1 = _bn_relu(h_ref[...], s1_ref[...], s2_ref[...], g_ref[...],
                      be_ref[...], n)
        h1_ref[...] = h1
        xs_ref[...] = h1 * dinv_ref[...]

    return pl.pallas_call(
        body,
        grid=(nb,),
        in_specs=[
            pl.BlockSpec((bn, d), lambda i: (i, 0)),
            pl.BlockSpec((nb, 8, d), lambda i: (0, 0, 0)),
            pl.BlockSpec((nb, 8, d), lambda i: (0, 0, 0)),
            pl.BlockSpec((1, d), lambda i: (0, 0)),
            pl.BlockSpec((1, d), lambda i: (0, 0)),
            pl.BlockSpec((bn, 1), lambda i: (i, 0)),
        ],
        out_specs=[
            pl.BlockSpec((bn, d), lambda i: (i, 0)),
            pl.BlockSpec((bn, d), lambda i: (i, 0)),
        ],
        out_shape=[
            jax.ShapeDtypeStruct((n, d), jnp.float32),
            jax.ShapeDtypeStruct((n, d), jnp.float32),
        ],
    )(h, s1, s2, g, be, dinv)


def _tc_bnrelu_final(h, s1, s2, g, be, h1, bn):
    n, d = h.shape
    nb = n // bn

    def body(h_ref, s1_ref, s2_ref, g_ref, be_ref, h1_ref, out_ref):
        h2 = _bn_relu(h_ref[...], s1_ref[...], s2_ref[...], g_ref[...],
                      be_ref[...], n)
        out_ref[...] = h1_ref[...] + h2

    return pl.pallas_call(
        body,
        grid=(nb,),
        in_specs=[
            pl.BlockSpec((bn, d), lambda i: (i, 0)),
            pl.BlockSpec((nb, 8, d), lambda i: (0, 0, 0)),
            pl.BlockSpec((nb, 8, d), lambda i: (0, 0, 0)),
            pl.BlockSpec((1, d), lambda i: (0, 0)),
            pl.BlockSpec((1, d), lambda i: (0, 0)),
            pl.BlockSpec((bn, d), lambda i: (i, 0)),
        ],
        out_specs=pl.BlockSpec((bn, d), lambda i: (i, 0)),
        out_shape=jax.ShapeDtypeStruct((n, d), jnp.float32),
    )(h, s1, s2, g, be, h1)


def kernel(x, edge_index, W1, b1, g1, be1, W2, b2, g2, be2):
    n, d = x.shape
    e = edge_index.shape[1]
    assert e % (_NW * _CH) == 0 and n % 200 == 0 and n % _CH == 0
    src1d = edge_index[0]
    dst1d = edge_index[1]
    b1r, g1r, be1r = b1.reshape(1, d), g1.reshape(1, d), be1.reshape(1, d)
    b2r, g2r, be2r = b2.reshape(1, d), g2.reshape(1, d), be2.reshape(1, d)
    bn = 1000

    degp = _sc_deg(dst1d, n)
    dinv, xs1 = _tc_prep(degp, x, bn)
    s1p = _sc_agg(xs1, src1d, dst1d)
    hpre1, a1, q1 = _tc_mmstats(s1p, x, dinv, W1, b1r, bn)
    h1, xs2 = _tc_bnrelu_mid(hpre1, a1, q1, g1r, be1r, dinv, bn)
    s2p = _sc_agg(xs2, src1d, dst1d)
    hpre2, a2, q2 = _tc_mmstats(s2p, h1, dinv, W2, b2r, bn)
    return _tc_bnrelu_final(hpre2, a2, q2, g2r, be2r, h1, bn)


# speedup vs baseline: 31.5506x; 1.1168x over previous
import functools

import jax
import jax.numpy as jnp
from jax import lax
from jax.experimental import pallas as pl
from jax.experimental.pallas import tpu as pltpu
from jax.experimental.pallas import tpu_sc as plsc

_NC = 2
_NS = 16
_NW = _NC * _NS
_CH = 80


def _fill_rows(ref, nrows, ncols, value):
    vec = jnp.full((16,), value, jnp.float32)

    def body(i, c):
        for g in range(ncols // 16):
            ref[i, 16 * g:16 * (g + 1)] = vec
        return c

    lax.fori_loop(0, nrows, body, 0)


def _sweep_chunks(n, dch, sid, fn):
    nck = n // dch
    npt = (nck + _NS - 1) // _NS

    def body(q, c):
        ck = sid + q * _NS

        @pl.when(ck < nck)
        def _():
            fn(ck * dch)

        return c

    lax.fori_loop(0, npt, body, 0)


def _sc_deg(dst1d, n):
    e = dst1d.shape[0]
    epw = e // _NW
    nit = epw // _CH
    mesh = plsc.VectorSubcoreMesh(core_axis_name="c", subcore_axis_name="s")

    @functools.partial(
        pl.kernel,
        out_type=jax.ShapeDtypeStruct((_NC, n, 16), jnp.float32),
        mesh=mesh,
        compiler_params=pltpu.CompilerParams(use_tc_tiling_on_sc=False),
        scratch_types=[
            pltpu.VMEM((e // _NW,), jnp.int32),
            pltpu.VMEM((_CH,), jnp.int32),
            pltpu.VMEM((_CH, 16), jnp.float32),
            pltpu.VMEM((200, 16), jnp.float32),
            pltpu.VMEM_SHARED((n, 16), jnp.float32),
        ],
    )
    def k(dst_hbm, out_hbm, flat_v, idx_v, ones_v, buf_v, acc_sh):
        cid = lax.axis_index("c")
        sid = lax.axis_index("s")
        wid = cid * _NS + sid
        _fill_rows(ones_v, _CH, 16, 1.0)
        _fill_rows(buf_v, 200, 16, 0.0)
        _sweep_chunks(n, 200, sid,
                      lambda r0: pltpu.sync_copy(
                          buf_v, acc_sh.at[pl.ds(r0, 200), :]))
        pltpu.sync_copy(dst_hbm.at[pl.ds(wid * epw, epw)], flat_v)
        plsc.subcore_barrier()

        def body(j, c):
            for c2 in range(_CH // 16):
                idx_v[16 * c2:16 * (c2 + 1)] = flat_v[pl.ds(j * _CH + 16 * c2,
                                                            16)]
            pltpu.sync_copy(ones_v, acc_sh.at[idx_v], add=True)
            return c

        lax.fori_loop(0, nit, body, 0)
        plsc.subcore_barrier()

        def drain(r0):
            pltpu.sync_copy(acc_sh.at[pl.ds(r0, 200), :], buf_v)
            pltpu.sync_copy(buf_v, out_hbm.at[cid, pl.ds(r0, 200), :])

        _sweep_chunks(n, 200, sid, drain)

    return k(dst1d)


def _sc_agg(xs, src1d, dst1d):
    n, d = xs.shape
    e = src1d.shape[0]
    epw = e // _NW
    nit = epw // _CH
    mesh = plsc.VectorSubcoreMesh(core_axis_name="c", subcore_axis_name="s")

    @functools.partial(
        pl.kernel,
        out_type=jax.ShapeDtypeStruct((_NC, n, d), jnp.float32),
        mesh=mesh,
        scratch_types=[
            pltpu.VMEM((e // _NW,), jnp.int32),
            [pltpu.VMEM((_CH,), jnp.int32) for _ in range(3)],
            [pltpu.VMEM((_CH, d), jnp.float32) for _ in range(3)],
            pltpu.VMEM_SHARED((n, d), jnp.float32),
            [pltpu.SemaphoreType.DMA for _ in range(3)],
            [pltpu.SemaphoreType.DMA for _ in range(3)],
            [pltpu.SemaphoreType.DMA for _ in range(3)],
        ],
    )
    def k(xs_hbm, src_hbm, dst_hbm, out_hbm, srcf_v, dstb, rows, acc_sh,
          semg, semi, semsc):
        cid = lax.axis_index("c")
        sid = lax.axis_index("s")
        wid = cid * _NS + sid
        _fill_rows(rows[0], _CH, d, 0.0)
        _sweep_chunks(n, _CH, sid,
                      lambda r0: pltpu.sync_copy(
                          rows[0], acc_sh.at[pl.ds(r0, _CH), :]))
        pltpu.sync_copy(src_hbm.at[pl.ds(wid * epw, epw)], srcf_v)
        plsc.subcore_barrier()

        def load_dst(j, s):
            pltpu.async_copy(dst_hbm.at[pl.ds(wid * epw + j * _CH, _CH)],
                             dstb[s], semi[s])

        def start_gather(j, s):
            pltpu.async_copy(xs_hbm.at[srcf_v.at[pl.ds(j * _CH, _CH)]],
                             rows[s], semg[s])

        def wait_gather(j, s):
            pltpu.make_async_copy(xs_hbm.at[srcf_v.at[pl.ds(j * _CH, _CH)]],
                                  rows[s], semg[s]).wait()
            pltpu.make_async_copy(dst_hbm.at[pl.ds(wid * epw + j * _CH, _CH)],
                                  dstb[s], semi[s]).wait()

        def wait_scatter(s):
            pltpu.make_async_copy(rows[s], acc_sh.at[dstb[s]],
                                  semsc[s]).wait()

        def turn(j, s, first_round):
            p = (s + 2) % 3
            wait_gather(j, s)
            pltpu.async_copy(rows[s], acc_sh.at[dstb[s]], semsc[s], add=True)

            def refill():
                if not first_round:
                    wait_scatter(p)
                load_dst(j + 2, p)
                start_gather(j + 2, p)

            if isinstance(j, int):
                if j + 2 < nit:
                    refill()
            else:
                pl.when(j + 2 < nit)(refill)

        load_dst(0, 0)
        start_gather(0, 0)
        load_dst(1, 1)
        start_gather(1, 1)

        def body0(q, c):
            turn(3 * q, 0, False)
            turn(3 * q + 1, 1, False)
            turn(3 * q + 2, 2, False)
            return c

        turn(0, 0, True)
        turn(1, 1, False)
        turn(2, 2, False)
        lax.fori_loop(1, nit // 3, body0, 0)
        for j in range(nit - (nit % 3), nit):
            turn(j, j % 3, False)
        for s in ((nit - 1) % 3, nit % 3, (nit + 1) % 3):
            wait_scatter(s)
        plsc.subcore_barrier()

        def drain(r0):
            pltpu.sync_copy(acc_sh.at[pl.ds(r0, _CH), :], rows[0])
            pltpu.sync_copy(rows[0], out_hbm.at[cid, pl.ds(r0, _CH), :])

        _sweep_chunks(n, _CH, sid, drain)

    return k(xs, src1d, dst1d)


def _tc_prep(degp, x, bn):
    n, d = x.shape

    def body(degp_ref, x_ref, dinv_ref, xs_ref):
        deg = degp_ref[0][:, 0:1] + degp_ref[1][:, 0:1] + 1.0
        dinv = lax.rsqrt(jnp.maximum(deg, 1.0))
        dinv_ref[...] = dinv
        xs_ref[...] = x_ref[...] * dinv

    return pl.pallas_call(
        body,
        grid=(n // bn,),
        in_specs=[
            pl.BlockSpec((_NC, bn, 16), lambda i: (0, i, 0)),
            pl.BlockSpec((bn, d), lambda i: (i, 0)),
        ],
        out_specs=[
            pl.BlockSpec((bn, 1), lambda i: (i, 0)),
            pl.BlockSpec((bn, d), lambda i: (i, 0)),
        ],
        out_shape=[
            jax.ShapeDtypeStruct((n, 1), jnp.float32),
            jax.ShapeDtypeStruct((n, d), jnp.float32),
        ],
    )(degp, x)


def _tc_mmstats(sp, xin, dinv, w, b, bn):
    n, d = xin.shape
    nb = n // bn

    def body(sp_ref, x_ref, dinv_ref, w_ref, b_ref, h_ref, s1_ref, s2_ref):
        dv = dinv_ref[...]
        agg = dv * (sp_ref[0] + sp_ref[1]) + (dv * dv) * x_ref[...]
        h = jnp.dot(agg, w_ref[...], preferred_element_type=jnp.float32)
        h = h + b_ref[...]
        h_ref[...] = h
        s1_ref[...] = jnp.broadcast_to(jnp.sum(h, axis=0, keepdims=True)[None],
                                       (1, 8, h.shape[1]))
        s2_ref[...] = jnp.broadcast_to(
            jnp.sum(h * h, axis=0, keepdims=True)[None], (1, 8, h.shape[1]))

    return pl.pallas_call(
        body,
        grid=(nb,),
        in_specs=[
            pl.BlockSpec((_NC, bn, d), lambda i: (0, i, 0)),
            pl.BlockSpec((bn, d), lambda i: (i, 0)),
            pl.BlockSpec((bn, 1), lambda i: (i, 0)),
            pl.BlockSpec((d, d), lambda i: (0, 0)),
            pl.BlockSpec((1, d), lambda i: (0, 0)),
        ],
        out_specs=[
            pl.BlockSpec((bn, d), lambda i: (i, 0)),
            pl.BlockSpec((1, 8, d), lambda i: (i, 0, 0)),
            pl.BlockSpec((1, 8, d), lambda i: (i, 0, 0)),
        ],
        out_shape=[
            jax.ShapeDtypeStruct((n, d), jnp.float32),
            jax.ShapeDtypeStruct((nb, 8, d), jnp.float32),
            jax.ShapeDtypeStruct((nb, 8, d), jnp.float32),
        ],
    )(sp, xin, dinv, w, b)


def _bn_relu(h, s1, s2, g, be, n):
    mu = jnp.sum(s1, axis=(0, 1))[None] * (1.0 / (8.0 * n))
    ex2 = jnp.sum(s2, axis=(0, 1))[None] * (1.0 / (8.0 * n))
    rstd = lax.rsqrt(jnp.maximum(ex2 - mu * mu, 0.0) + 1e-5)
    return jnp.maximum((h - mu) * rstd * g + be, 0.0)


def _tc_bnrelu_mid(h, s1, s2, g, be, dinv, bn):
    n, d = h.shape
    nb = n // bn

    def body(h_ref, s1_ref, s2_ref, g_ref, be_ref, dinv_ref, h1_ref, xs_ref):
        h1 = _bn_relu(h_ref[...], s1_ref[...], s2_ref[...], g_ref[...],
                      be_ref[...], n)
        h1_ref[...] = h1
        xs_ref[...] = h1 * dinv_ref[...]

    return pl.pallas_call(
        body,
        grid=(nb,),
        in_specs=[
            pl.BlockSpec((bn, d), lambda i: (i, 0)),
            pl.BlockSpec((nb, 8, d), lambda i: (0, 0, 0)),
            pl.BlockSpec((nb, 8, d), lambda i: (0, 0, 0)),
            pl.BlockSpec((1, d), lambda i: (0, 0)),
            pl.BlockSpec((1, d), lambda i: (0, 0)),
            pl.BlockSpec((bn, 1), lambda i: (i, 0)),
        ],
        out_specs=[
            pl.BlockSpec((bn, d), lambda i: (i, 0)),
            pl.BlockSpec((bn, d), lambda i: (i, 0)),
        ],
        out_shape=[
            jax.ShapeDtypeStruct((n, d), jnp.float32),
            jax.ShapeDtypeStruct((n, d), jnp.float32),
        ],
    )(h, s1, s2, g, be, dinv)


def _tc_bnrelu_final(h, s1, s2, g, be, h1, bn):
    n, d = h.shape
    nb = n // bn

    def body(h_ref, s1_ref, s2_ref, g_ref, be_ref, h1_ref, out_ref):
        h2 = _bn_relu(h_ref[...], s1_ref[...], s2_ref[...], g_ref[...],
                      be_ref[...], n)
        out_ref[...] = h1_ref[...] + h2

    return pl.pallas_call(
        body,
        grid=(nb,),
        in_specs=[
            pl.BlockSpec((bn, d), lambda i: (i, 0)),
            pl.BlockSpec((nb, 8, d), lambda i: (0, 0, 0)),
            pl.BlockSpec((nb, 8, d), lambda i: (0, 0, 0)),
            pl.BlockSpec((1, d), lambda i: (0, 0)),
            pl.BlockSpec((1, d), lambda i: (0, 0)),
            pl.BlockSpec((bn, d), lambda i: (i, 0)),
        ],
        out_specs=pl.BlockSpec((bn, d), lambda i: (i, 0)),
        out_shape=jax.ShapeDtypeStruct((n, d), jnp.float32),
    )(h, s1, s2, g, be, h1)


def kernel(x, edge_index, W1, b1, g1, be1, W2, b2, g2, be2):
    n, d = x.shape
    e = edge_index.shape[1]
    assert e % (_NW * _CH) == 0 and n % 200 == 0 and n % _CH == 0
    src1d = edge_index[0]
    dst1d = edge_index[1]
    b1r, g1r, be1r = b1.reshape(1, d), g1.reshape(1, d), be1.reshape(1, d)
    b2r, g2r, be2r = b2.reshape(1, d), g2.reshape(1, d), be2.reshape(1, d)
    bn = 1000

    degp = _sc_deg(dst1d, n)
    dinv, xs1 = _tc_prep(degp, x, bn)
    s1p = _sc_agg(xs1, src1d, dst1d)
    hpre1, a1, q1 = _tc_mmstats(s1p, x, dinv, W1, b1r, bn)
    h1, xs2 = _tc_bnrelu_mid(hpre1, a1, q1, g1r, be1r, dinv, bn)
    s2p = _sc_agg(xs2, src1d, dst1d)
    hpre2, a2, q2 = _tc_mmstats(s2p, h1, dinv, W2, b2r, bn)
    return _tc_bnrelu_final(hpre2, a2, q2, g2r, be2r, h1, bn)


# speedup vs baseline: 32.1753x; 1.0198x over previous
import functools

import jax
import jax.numpy as jnp
from jax import lax
from jax.experimental import pallas as pl
from jax.experimental.pallas import tpu as pltpu
from jax.experimental.pallas import tpu_sc as plsc

_NC = 2
_NS = 16
_NW = _NC * _NS
_CH = 80


def _fill_rows(ref, nrows, ncols, value):
    vec = jnp.full((16,), value, jnp.float32)

    def body(i, c):
        for g in range(ncols // 16):
            ref[i, 16 * g:16 * (g + 1)] = vec
        return c

    lax.fori_loop(0, nrows, body, 0)


def _sweep_chunks(n, dch, sid, fn):
    nck = n // dch
    npt = (nck + _NS - 1) // _NS

    def body(q, c):
        ck = sid + q * _NS

        @pl.when(ck < nck)
        def _():
            fn(ck * dch)

        return c

    lax.fori_loop(0, npt, body, 0)


def _sc_deg(dst1d, n):
    e = dst1d.shape[0]
    epw = e // _NW
    nit = epw // _CH
    mesh = plsc.VectorSubcoreMesh(core_axis_name="c", subcore_axis_name="s")

    @functools.partial(
        pl.kernel,
        out_type=jax.ShapeDtypeStruct((_NC, n, 16), jnp.float32),
        mesh=mesh,
        compiler_params=pltpu.CompilerParams(use_tc_tiling_on_sc=False),
        scratch_types=[
            pltpu.VMEM((e // _NW,), jnp.int32),
            pltpu.VMEM((_CH,), jnp.int32),
            pltpu.VMEM((_CH, 16), jnp.float32),
            pltpu.VMEM((200, 16), jnp.float32),
            pltpu.VMEM_SHARED((n, 16), jnp.float32),
        ],
    )
    def k(dst_hbm, out_hbm, flat_v, idx_v, ones_v, buf_v, acc_sh):
        cid = lax.axis_index("c")
        sid = lax.axis_index("s")
        wid = cid * _NS + sid
        _fill_rows(ones_v, _CH, 16, 1.0)
        _fill_rows(buf_v, 200, 16, 0.0)
        _sweep_chunks(n, 200, sid,
                      lambda r0: pltpu.sync_copy(
                          buf_v, acc_sh.at[pl.ds(r0, 200), :]))
        pltpu.sync_copy(dst_hbm.at[pl.ds(wid * epw, epw)], flat_v)
        plsc.subcore_barrier()

        def body(j, c):
            for c2 in range(_CH // 16):
                idx_v[16 * c2:16 * (c2 + 1)] = flat_v[pl.ds(j * _CH + 16 * c2,
                                                            16)]
            pltpu.sync_copy(ones_v, acc_sh.at[idx_v], add=True)
            return c

        lax.fori_loop(0, nit, body, 0)
        plsc.subcore_barrier()

        def drain(r0):
            pltpu.sync_copy(acc_sh.at[pl.ds(r0, 200), :], buf_v)
            pltpu.sync_copy(buf_v, out_hbm.at[cid, pl.ds(r0, 200), :])

        _sweep_chunks(n, 200, sid, drain)

    return k(dst1d)


def _sc_agg(xs, src1d, dst1d):
    n, d = xs.shape
    e = src1d.shape[0]
    epw = e // _NW
    nit = epw // _CH
    mesh = plsc.VectorSubcoreMesh(core_axis_name="c", subcore_axis_name="s")

    @functools.partial(
        pl.kernel,
        out_type=jax.ShapeDtypeStruct((_NC, n, d), jnp.float32),
        mesh=mesh,
        scratch_types=[
            pltpu.VMEM((e // _NW,), jnp.int32),
            [pltpu.VMEM((_CH,), jnp.int32) for _ in range(3)],
            [pltpu.VMEM((_CH, d), jnp.float32) for _ in range(3)],
            pltpu.VMEM_SHARED((n, d), jnp.float32),
            [pltpu.SemaphoreType.DMA for _ in range(3)],
            [pltpu.SemaphoreType.DMA for _ in range(3)],
            [pltpu.SemaphoreType.DMA for _ in range(3)],
        ],
    )
    def k(xs_hbm, src_hbm, dst_hbm, out_hbm, srcf_v, dstb, rows, acc_sh,
          semg, semi, semsc):
        cid = lax.axis_index("c")
        sid = lax.axis_index("s")
        wid = cid * _NS + sid
        _fill_rows(rows[0], _CH, d, 0.0)
        _sweep_chunks(n, _CH, sid,
                      lambda r0: pltpu.sync_copy(
                          rows[0], acc_sh.at[pl.ds(r0, _CH), :]))
        pltpu.sync_copy(src_hbm.at[pl.ds(wid * epw, epw)], srcf_v)
        plsc.subcore_barrier()

        def load_dst(j, s):
            pltpu.async_copy(dst_hbm.at[pl.ds(wid * epw + j * _CH, _CH)],
                             dstb[s], semi[s])

        def start_gather(j, s):
            pltpu.async_copy(xs_hbm.at[srcf_v.at[pl.ds(j * _CH, _CH)]],
                             rows[s], semg[s])

        def wait_gather(j, s):
            pltpu.make_async_copy(xs_hbm.at[srcf_v.at[pl.ds(j * _CH, _CH)]],
                                  rows[s], semg[s]).wait()
            pltpu.make_async_copy(dst_hbm.at[pl.ds(wid * epw + j * _CH, _CH)],
                                  dstb[s], semi[s]).wait()

        def wait_scatter(s):
            pltpu.make_async_copy(rows[s], acc_sh.at[dstb[s]],
                                  semsc[s]).wait()

        def turn(j, s, first_round):
            p = (s + 2) % 3
            wait_gather(j, s)
            pltpu.async_copy(rows[s], acc_sh.at[dstb[s]], semsc[s], add=True)

            def refill():
                if not first_round:
                    wait_scatter(p)
                load_dst(j + 2, p)
                start_gather(j + 2, p)

            if isinstance(j, int):
                if j + 2 < nit:
                    refill()
            else:
                pl.when(j + 2 < nit)(refill)

        load_dst(0, 0)
        start_gather(0, 0)
        load_dst(1, 1)
        start_gather(1, 1)

        def body0(q, c):
            turn(3 * q, 0, False)
            turn(3 * q + 1, 1, False)
            turn(3 * q + 2, 2, False)
            return c

        turn(0, 0, True)
        turn(1, 1, False)
        turn(2, 2, False)
        lax.fori_loop(1, nit // 3, body0, 0)
        for j in range(nit - (nit % 3), nit):
            turn(j, j % 3, False)
        for s in ((nit - 1) % 3, nit % 3, (nit + 1) % 3):
            wait_scatter(s)
        plsc.subcore_barrier()

        def drain(r0):
            pltpu.sync_copy(acc_sh.at[pl.ds(r0, _CH), :], rows[0])
            pltpu.sync_copy(rows[0], out_hbm.at[cid, pl.ds(r0, _CH), :])

        _sweep_chunks(n, _CH, sid, drain)

    return k(xs, src1d, dst1d)


def _tc_prep(degp, x, bn):
    n, d = x.shape

    def body(degp_ref, x_ref, dinv_ref, xs_ref):
        deg = degp_ref[0][:, 0:1] + degp_ref[1][:, 0:1] + 1.0
        dinv = lax.rsqrt(jnp.maximum(deg, 1.0))
        dinv_ref[...] = dinv
        xs_ref[...] = x_ref[...] * dinv

    return pl.pallas_call(
        body,
        grid=(n // bn,),
        in_specs=[
            pl.BlockSpec((_NC, bn, 16), lambda i: (0, i, 0)),
            pl.BlockSpec((bn, d), lambda i: (i, 0)),
        ],
        out_specs=[
            pl.BlockSpec((bn, 1), lambda i: (i, 0)),
            pl.BlockSpec((bn, d), lambda i: (i, 0)),
        ],
        out_shape=[
            jax.ShapeDtypeStruct((n, 1), jnp.float32),
            jax.ShapeDtypeStruct((n, d), jnp.float32),
        ],
    )(degp, x)


def _tc_layer(sp, xin, dinv, w, b, g, be, h1prev, bn):
    n, d = xin.shape
    nb = n // bn
    mid = h1prev is None

    def body(*refs):
        if mid:
            (sp_ref, x_ref, dinv_ref, w_ref, b_ref, g_ref, be_ref,
             o1_ref, o2_ref, hs, s1a, s2a) = refs
        else:
            (sp_ref, x_ref, dinv_ref, w_ref, b_ref, g_ref, be_ref, hp_ref,
             o1_ref, hs, s1a, s2a) = refs
        i = pl.program_id(0)

        @pl.when(i < nb)
        def _():
            dv = dinv_ref[...]
            agg = dv * (sp_ref[0] + sp_ref[1]) + (dv * dv) * x_ref[...]
            h = jnp.dot(agg, w_ref[...], preferred_element_type=jnp.float32)
            h = h + b_ref[...]
            hs[pl.ds(i * bn, bn), :] = h
            s1 = jnp.sum(h, axis=0, keepdims=True)
            s2 = jnp.sum(h * h, axis=0, keepdims=True)

            @pl.when(i == 0)
            def _():
                s1a[...] = s1
                s2a[...] = s2

            @pl.when(i > 0)
            def _():
                s1a[...] += s1
                s2a[...] += s2

        @pl.when(i >= nb)
        def _():
            mu = s1a[...] * (1.0 / n)
            ex2 = s2a[...] * (1.0 / n)
            rstd = lax.rsqrt(jnp.maximum(ex2 - mu * mu, 0.0) + 1e-5)
            hblk = hs[pl.ds((i - nb) * bn, bn), :]
            h1 = jnp.maximum((hblk - mu) * rstd * g_ref[...] + be_ref[...],
                             0.0)
            if mid:
                o1_ref[...] = h1
                o2_ref[...] = h1 * dinv_ref[...]
            else:
                o1_ref[...] = hp_ref[...] + h1

    lo = lambda i: (jnp.where(i < nb, i, 0), 0)
    hi = lambda i: (jnp.where(i < nb, 0, i - nb), 0)
    in_specs = [
        pl.BlockSpec((_NC, bn, d), lambda i: (0, jnp.where(i < nb, i, 0), 0)),
        pl.BlockSpec((bn, d), lo),
        pl.BlockSpec((bn, 1), lambda i: (i % nb, 0) if mid else lo(i)),
        pl.BlockSpec((d, d), lambda i: (0, 0)),
        pl.BlockSpec((1, d), lambda i: (0, 0)),
        pl.BlockSpec((1, d), lambda i: (0, 0)),
        pl.BlockSpec((1, d), lambda i: (0, 0)),
    ]
    args = [sp, xin, dinv, w, b, g, be]
    if mid:
        out_specs = [pl.BlockSpec((bn, d), hi), pl.BlockSpec((bn, d), hi)]
        out_shape = [jax.ShapeDtypeStruct((n, d), jnp.float32)] * 2
    else:
        in_specs.append(pl.BlockSpec((bn, d), hi))
        args.append(h1prev)
        out_specs = pl.BlockSpec((bn, d), hi)
        out_shape = jax.ShapeDtypeStruct((n, d), jnp.float32)

    return pl.pallas_call(
        body,
        grid=(2 * nb,),
        in_specs=in_specs,
        out_specs=out_specs,
        out_shape=out_shape,
        scratch_shapes=[
            pltpu.VMEM((n, d), jnp.float32),
            pltpu.VMEM((1, d), jnp.float32),
            pltpu.VMEM((1, d), jnp.float32),
        ],
    )(*args)


def kernel(x, edge_index, W1, b1, g1, be1, W2, b2, g2, be2):
    n, d = x.shape
    e = edge_index.shape[1]
    assert e % (_NW * _CH) == 0 and n % 200 == 0 and n % _CH == 0
    src1d = edge_index[0]
    dst1d = edge_index[1]
    b1r, g1r, be1r = b1.reshape(1, d), g1.reshape(1, d), be1.reshape(1, d)
    b2r, g2r, be2r = b2.reshape(1, d), g2.reshape(1, d), be2.reshape(1, d)
    bn = 1000

    degp = _sc_deg(dst1d, n)
    dinv, xs1 = _tc_prep(degp, x, bn)
    s1p = _sc_agg(xs1, src1d, dst1d)
    h1, xs2 = _tc_layer(s1p, x, dinv, W1, b1r, g1r, be1r, None, bn)
    s2p = _sc_agg(xs2, src1d, dst1d)
    return _tc_layer(s2p, h1, dinv, W2, b2r, g2r, be2r, h1, bn)


# speedup vs baseline: 32.6682x; 1.0153x over previous
import functools

import jax
import jax.numpy as jnp
from jax import lax
from jax.experimental import pallas as pl
from jax.experimental.pallas import tpu as pltpu
from jax.experimental.pallas import tpu_sc as plsc

_NC = 2
_NS = 16
_NW = _NC * _NS
_CH = 80


def _fill_rows(ref, nrows, ncols, value):
    vec = jnp.full((16,), value, jnp.float32)

    def body(i, c):
        for g in range(ncols // 16):
            ref[i, 16 * g:16 * (g + 1)] = vec
        return c

    lax.fori_loop(0, nrows, body, 0)


def _sweep_chunks(n, dch, sid, fn):
    nck = n // dch
    npt = (nck + _NS - 1) // _NS

    def body(q, c):
        ck = sid + q * _NS

        @pl.when(ck < nck)
        def _():
            fn(ck * dch)

        return c

    lax.fori_loop(0, npt, body, 0)


def _sc_deg(dst1d, n):
    e = dst1d.shape[0]
    epw = e // _NW
    nit = epw // _CH
    mesh = plsc.VectorSubcoreMesh(core_axis_name="c", subcore_axis_name="s")

    @functools.partial(
        pl.kernel,
        out_type=jax.ShapeDtypeStruct((_NC, n, 16), jnp.float32),
        mesh=mesh,
        compiler_params=pltpu.CompilerParams(use_tc_tiling_on_sc=False),
        scratch_types=[
            pltpu.VMEM((e // _NW,), jnp.int32),
            pltpu.VMEM((_CH,), jnp.int32),
            pltpu.VMEM((_CH, 16), jnp.float32),
            pltpu.VMEM((200, 16), jnp.float32),
            pltpu.VMEM_SHARED((n, 16), jnp.float32),
        ],
    )
    def k(dst_hbm, out_hbm, flat_v, idx_v, ones_v, buf_v, acc_sh):
        cid = lax.axis_index("c")
        sid = lax.axis_index("s")
        wid = cid * _NS + sid
        _fill_rows(ones_v, _CH, 16, 1.0)
        _fill_rows(buf_v, 200, 16, 0.0)
        _sweep_chunks(n, 200, sid,
                      lambda r0: pltpu.sync_copy(
                          buf_v, acc_sh.at[pl.ds(r0, 200), :]))
        pltpu.sync_copy(dst_hbm.at[pl.ds(wid * epw, epw)], flat_v)
        plsc.subcore_barrier()

        def body(j, c):
            for c2 in range(_CH // 16):
                idx_v[16 * c2:16 * (c2 + 1)] = flat_v[pl.ds(j * _CH + 16 * c2,
                                                            16)]
            pltpu.sync_copy(ones_v, acc_sh.at[idx_v], add=True)
            return c

        lax.fori_loop(0, nit, body, 0)
        plsc.subcore_barrier()

        def drain(r0):
            pltpu.sync_copy(acc_sh.at[pl.ds(r0, 200), :],
                            out_hbm.at[cid, pl.ds(r0, 200), :])

        _sweep_chunks(n, 200, sid, drain)

    return k(dst1d)


def _sc_agg(xs, src1d, dst1d):
    n, d = xs.shape
    e = src1d.shape[0]
    epw = e // _NW
    nit = epw // _CH
    mesh = plsc.VectorSubcoreMesh(core_axis_name="c", subcore_axis_name="s")

    @functools.partial(
        pl.kernel,
        out_type=jax.ShapeDtypeStruct((_NC, n, d), jnp.float32),
        mesh=mesh,
        scratch_types=[
            pltpu.VMEM((e // _NW,), jnp.int32),
            [pltpu.VMEM((_CH,), jnp.int32) for _ in range(3)],
            [pltpu.VMEM((_CH, d), jnp.float32) for _ in range(3)],
            pltpu.VMEM_SHARED((n, d), jnp.float32),
            [pltpu.SemaphoreType.DMA for _ in range(3)],
            [pltpu.SemaphoreType.DMA for _ in range(3)],
            [pltpu.SemaphoreType.DMA for _ in range(3)],
        ],
    )
    def k(xs_hbm, src_hbm, dst_hbm, out_hbm, srcf_v, dstb, rows, acc_sh,
          semg, semi, semsc):
        cid = lax.axis_index("c")
        sid = lax.axis_index("s")
        wid = cid * _NS + sid
        _fill_rows(rows[0], _CH, d, 0.0)

        def zero(r0):
            pltpu.async_copy(rows[0], acc_sh.at[pl.ds(r0, _CH), :], semsc[0])

        _sweep_chunks(n, _CH, sid, zero)
        pltpu.sync_copy(src_hbm.at[pl.ds(wid * epw, epw)], srcf_v)
        _sweep_chunks(n, _CH, sid,
                      lambda r0: pltpu.make_async_copy(
                          rows[0], acc_sh.at[pl.ds(r0, _CH), :],
                          semsc[0]).wait())
        plsc.subcore_barrier()

        def load_dst(j, s):
            pltpu.async_copy(dst_hbm.at[pl.ds(wid * epw + j * _CH, _CH)],
                             dstb[s], semi[s])

        def start_gather(j, s):
            pltpu.async_copy(xs_hbm.at[srcf_v.at[pl.ds(j * _CH, _CH)]],
                             rows[s], semg[s])

        def wait_gather(j, s):
            pltpu.make_async_copy(xs_hbm.at[srcf_v.at[pl.ds(j * _CH, _CH)]],
                                  rows[s], semg[s]).wait()
            pltpu.make_async_copy(dst_hbm.at[pl.ds(wid * epw + j * _CH, _CH)],
                                  dstb[s], semi[s]).wait()

        def wait_scatter(s):
            pltpu.make_async_copy(rows[s], acc_sh.at[dstb[s]],
                                  semsc[s]).wait()

        def turn(j, s, first_round):
            p = (s + 2) % 3
            wait_gather(j, s)
            pltpu.async_copy(rows[s], acc_sh.at[dstb[s]], semsc[s], add=True)

            def refill():
                if not first_round:
                    wait_scatter(p)
                load_dst(j + 2, p)
                start_gather(j + 2, p)

            if isinstance(j, int):
                if j + 2 < nit:
                    refill()
            else:
                pl.when(j + 2 < nit)(refill)

        load_dst(0, 0)
        start_gather(0, 0)
        load_dst(1, 1)
        start_gather(1, 1)

        def body0(q, c):
            turn(3 * q, 0, False)
            turn(3 * q + 1, 1, False)
            turn(3 * q + 2, 2, False)
            return c

        turn(0, 0, True)
        turn(1, 1, False)
        turn(2, 2, False)
        lax.fori_loop(1, nit // 3, body0, 0)
        for j in range(nit - (nit % 3), nit):
            turn(j, j % 3, False)
        for s in ((nit - 1) % 3, nit % 3, (nit + 1) % 3):
            wait_scatter(s)
        plsc.subcore_barrier()

        def drain(r0):
            pltpu.sync_copy(acc_sh.at[pl.ds(r0, _CH), :],
                            out_hbm.at[cid, pl.ds(r0, _CH), :])

        _sweep_chunks(n, _CH, sid, drain)

    return k(xs, src1d, dst1d)


def _tc_prep(degp, x, bn):
    n, d = x.shape

    def body(degp_ref, x_ref, dinv_ref, xs_ref):
        deg = degp_ref[0][:, 0:1] + degp_ref[1][:, 0:1] + 1.0
        dinv = lax.rsqrt(jnp.maximum(deg, 1.0))
        dinv_ref[...] = dinv
        xs_ref[...] = x_ref[...] * dinv

    return pl.pallas_call(
        body,
        grid=(n // bn,),
        in_specs=[
            pl.BlockSpec((_NC, bn, 16), lambda i: (0, i, 0)),
            pl.BlockSpec((bn, d), lambda i: (i, 0)),
        ],
        out_specs=[
            pl.BlockSpec((bn, 1), lambda i: (i, 0)),
            pl.BlockSpec((bn, d), lambda i: (i, 0)),
        ],
        out_shape=[
            jax.ShapeDtypeStruct((n, 1), jnp.float32),
            jax.ShapeDtypeStruct((n, d), jnp.float32),
        ],
    )(degp, x)


def _tc_layer(sp, xin, dinv, w, b, g, be, h1prev, bn):
    n, d = xin.shape
    nb = n // bn
    mid = h1prev is None

    def body(*refs):
        if mid:
            (sp_ref, x_ref, dinv_ref, w_ref, b_ref, g_ref, be_ref,
             o1_ref, o2_ref, hs, s1a, s2a) = refs
        else:
            (sp_ref, x_ref, dinv_ref, w_ref, b_ref, g_ref, be_ref, hp_ref,
             o1_ref, hs, s1a, s2a) = refs
        i = pl.program_id(0)

        @pl.when(i < nb)
        def _():
            dv = dinv_ref[...]
            agg = dv * (sp_ref[0] + sp_ref[1]) + (dv * dv) * x_ref[...]
            h = jnp.dot(agg, w_ref[...], preferred_element_type=jnp.float32)
            h = h + b_ref[...]
            hs[pl.ds(i * bn, bn), :] = h
            s1 = jnp.sum(h, axis=0, keepdims=True)
            s2 = jnp.sum(h * h, axis=0, keepdims=True)

            @pl.when(i == 0)
            def _():
                s1a[...] = s1
                s2a[...] = s2

            @pl.when(i > 0)
            def _():
                s1a[...] += s1
                s2a[...] += s2

        @pl.when(i >= nb)
        def _():
            mu = s1a[...] * (1.0 / n)
            ex2 = s2a[...] * (1.0 / n)
            rstd = lax.rsqrt(jnp.maximum(ex2 - mu * mu, 0.0) + 1e-5)
            hblk = hs[pl.ds((i - nb) * bn, bn), :]
            h1 = jnp.maximum((hblk - mu) * rstd * g_ref[...] + be_ref[...],
                             0.0)
            if mid:
                o1_ref[...] = h1
                o2_ref[...] = h1 * dinv_ref[...]
            else:
                o1_ref[...] = hp_ref[...] + h1

    lo = lambda i: (jnp.where(i < nb, i, 0), 0)
    hi = lambda i: (jnp.where(i < nb, 0, i - nb), 0)
    in_specs = [
        pl.BlockSpec((_NC, bn, d), lambda i: (0, jnp.where(i < nb, i, 0), 0)),
        pl.BlockSpec((bn, d), lo),
        pl.BlockSpec((bn, 1), lambda i: (i % nb, 0) if mid else lo(i)),
        pl.BlockSpec((d, d), lambda i: (0, 0)),
        pl.BlockSpec((1, d), lambda i: (0, 0)),
        pl.BlockSpec((1, d), lambda i: (0, 0)),
        pl.BlockSpec((1, d), lambda i: (0, 0)),
    ]
    args = [sp, xin, dinv, w, b, g, be]
    if mid:
        out_specs = [pl.BlockSpec((bn, d), hi), pl.BlockSpec((bn, d), hi)]
        out_shape = [jax.ShapeDtypeStruct((n, d), jnp.float32)] * 2
    else:
        in_specs.append(pl.BlockSpec((bn, d), hi))
        args.append(h1prev)
        out_specs = pl.BlockSpec((bn, d), hi)
        out_shape = jax.ShapeDtypeStruct((n, d), jnp.float32)

    return pl.pallas_call(
        body,
        grid=(2 * nb,),
        in_specs=in_specs,
        out_specs=out_specs,
        out_shape=out_shape,
        scratch_shapes=[
            pltpu.VMEM((n, d), jnp.float32),
            pltpu.VMEM((1, d), jnp.float32),
            pltpu.VMEM((1, d), jnp.float32),
        ],
    )(*args)


def kernel(x, edge_index, W1, b1, g1, be1, W2, b2, g2, be2):
    n, d = x.shape
    e = edge_index.shape[1]
    assert e % (_NW * _CH) == 0 and n % 200 == 0 and n % _CH == 0
    src1d = edge_index[0]
    dst1d = edge_index[1]
    b1r, g1r, be1r = b1.reshape(1, d), g1.reshape(1, d), be1.reshape(1, d)
    b2r, g2r, be2r = b2.reshape(1, d), g2.reshape(1, d), be2.reshape(1, d)
    bn = 1000

    degp = _sc_deg(dst1d, n)
    dinv, xs1 = _tc_prep(degp, x, bn)
    s1p = _sc_agg(xs1, src1d, dst1d)
    h1, xs2 = _tc_layer(s1p, x, dinv, W1, b1r, g1r, be1r, None, bn)
    s2p = _sc_agg(xs2, src1d, dst1d)
    return _tc_layer(s2p, h1, dinv, W2, b2r, g2r, be2r, h1, bn)


# speedup vs baseline: 33.3622x; 1.0212x over previous
import functools

import jax
import jax.numpy as jnp
from jax import lax
from jax.experimental import pallas as pl
from jax.experimental.pallas import tpu as pltpu
from jax.experimental.pallas import tpu_sc as plsc

_NC = 2
_NS = 16
_NW = _NC * _NS
_CH = 80


def _fill_rows(ref, nrows, ncols, value):
    vec = jnp.full((16,), value, jnp.float32)

    def body(i, c):
        for g in range(ncols // 16):
            ref[i, 16 * g:16 * (g + 1)] = vec
        return c

    lax.fori_loop(0, nrows, body, 0)


def _sweep_chunks(n, dch, sid, fn):
    nck = n // dch
    npt = (nck + _NS - 1) // _NS

    def body(q, c):
        ck = sid + q * _NS

        @pl.when(ck < nck)
        def _():
            fn(ck * dch)

        return c

    lax.fori_loop(0, npt, body, 0)


def _sc_deg(dst1d, n):
    e = dst1d.shape[0]
    epw = e // _NW
    nit = epw // _CH
    mesh = plsc.VectorSubcoreMesh(core_axis_name="c", subcore_axis_name="s")

    @functools.partial(
        pl.kernel,
        out_type=jax.ShapeDtypeStruct((_NC, n, 16), jnp.float32),
        mesh=mesh,
        compiler_params=pltpu.CompilerParams(use_tc_tiling_on_sc=False),
        scratch_types=[
            pltpu.VMEM((e // _NW,), jnp.int32),
            [pltpu.VMEM((_CH,), jnp.int32) for _ in range(2)],
            pltpu.VMEM((_CH, 16), jnp.float32),
            pltpu.VMEM((200, 16), jnp.float32),
            pltpu.VMEM_SHARED((n, 16), jnp.float32),
            [pltpu.SemaphoreType.DMA for _ in range(2)],
        ],
    )
    def k(dst_hbm, out_hbm, flat_v, idxb, ones_v, buf_v, acc_sh, semsc):
        cid = lax.axis_index("c")
        sid = lax.axis_index("s")
        wid = cid * _NS + sid
        _fill_rows(ones_v, _CH, 16, 1.0)
        _fill_rows(buf_v, 200, 16, 0.0)
        _sweep_chunks(n, 200, sid,
                      lambda r0: pltpu.sync_copy(
                          buf_v, acc_sh.at[pl.ds(r0, 200), :]))
        pltpu.sync_copy(dst_hbm.at[pl.ds(wid * epw, epw)], flat_v)
        plsc.subcore_barrier()

        def fill_idx(j, s):
            for c2 in range(_CH // 16):
                idxb[s][16 * c2:16 * (c2 + 1)] = flat_v[
                    pl.ds(j * _CH + 16 * c2, 16)]

        def wait_scatter(s):
            pltpu.make_async_copy(ones_v, acc_sh.at[idxb[s]],
                                  semsc[s]).wait()

        def scat(s):
            pltpu.async_copy(ones_v, acc_sh.at[idxb[s]], semsc[s], add=True)

        fill_idx(0, 0)
        scat(0)

        def body(q, c):
            j0 = 2 * q
            fill_idx(j0 + 1, 1)
            scat(1)
            wait_scatter(0)

            @pl.when(j0 + 2 < nit)
            def _():
                fill_idx(j0 + 2, 0)
                scat(0)

            wait_scatter(1)
            return c

        lax.fori_loop(0, nit // 2, body, 0)
        if nit % 2:
            wait_scatter(0)
        plsc.subcore_barrier()

        def drain(r0):
            pltpu.sync_copy(acc_sh.at[pl.ds(r0, 200), :],
                            out_hbm.at[cid, pl.ds(r0, 200), :])

        _sweep_chunks(n, 200, sid, drain)

    return k(dst1d)


def _sc_agg(xs, src1d, dst1d):
    n, d = xs.shape
    e = src1d.shape[0]
    epw = e // _NW
    nit = epw // _CH
    mesh = plsc.VectorSubcoreMesh(core_axis_name="c", subcore_axis_name="s")

    @functools.partial(
        pl.kernel,
        out_type=jax.ShapeDtypeStruct((_NC, n, d), jnp.float32),
        mesh=mesh,
        scratch_types=[
            pltpu.VMEM((e // _NW,), jnp.int32),
            [pltpu.VMEM((_CH,), jnp.int32) for _ in range(3)],
            [pltpu.VMEM((_CH, d), jnp.float32) for _ in range(3)],
            pltpu.VMEM_SHARED((n, d), jnp.float32),
            [pltpu.SemaphoreType.DMA for _ in range(3)],
            [pltpu.SemaphoreType.DMA for _ in range(3)],
            [pltpu.SemaphoreType.DMA for _ in range(3)],
        ],
    )
    def k(xs_hbm, src_hbm, dst_hbm, out_hbm, srcf_v, dstb, rows, acc_sh,
          semg, semi, semsc):
        cid = lax.axis_index("c")
        sid = lax.axis_index("s")
        wid = cid * _NS + sid
        _fill_rows(rows[0], _CH, d, 0.0)

        def zero(r0):
            pltpu.async_copy(rows[0], acc_sh.at[pl.ds(r0, _CH), :], semsc[0])

        _sweep_chunks(n, _CH, sid, zero)
        pltpu.sync_copy(src_hbm.at[pl.ds(wid * epw, epw)], srcf_v)
        _sweep_chunks(n, _CH, sid,
                      lambda r0: pltpu.make_async_copy(
                          rows[0], acc_sh.at[pl.ds(r0, _CH), :],
                          semsc[0]).wait())
        plsc.subcore_barrier()

        def load_dst(j, s):
            pltpu.async_copy(dst_hbm.at[pl.ds(wid * epw + j * _CH, _CH)],
                             dstb[s], semi[s])

        def start_gather(j, s):
            pltpu.async_copy(xs_hbm.at[srcf_v.at[pl.ds(j * _CH, _CH)]],
                             rows[s], semg[s])

        def wait_gather(j, s):
            pltpu.make_async_copy(xs_hbm.at[srcf_v.at[pl.ds(j * _CH, _CH)]],
                                  rows[s], semg[s]).wait()
            pltpu.make_async_copy(dst_hbm.at[pl.ds(wid * epw + j * _CH, _CH)],
                                  dstb[s], semi[s]).wait()

        def wait_scatter(s):
            pltpu.make_async_copy(rows[s], acc_sh.at[dstb[s]],
                                  semsc[s]).wait()

        def turn(j, s, first_round):
            p = (s + 2) % 3
            wait_gather(j, s)
            pltpu.async_copy(rows[s], acc_sh.at[dstb[s]], semsc[s], add=True)

            def refill():
                if not first_round:
                    wait_scatter(p)
                load_dst(j + 2, p)
                start_gather(j + 2, p)

            if isinstance(j, int):
                if j + 2 < nit:
                    refill()
            else:
                pl.when(j + 2 < nit)(refill)

        load_dst(0, 0)
        start_gather(0, 0)
        load_dst(1, 1)
        start_gather(1, 1)

        def body0(q, c):
            turn(3 * q, 0, False)
            turn(3 * q + 1, 1, False)
            turn(3 * q + 2, 2, False)
            return c

        turn(0, 0, True)
        turn(1, 1, False)
        turn(2, 2, False)
        lax.fori_loop(1, nit // 3, body0, 0)
        for j in range(nit - (nit % 3), nit):
            turn(j, j % 3, False)
        for s in ((nit - 1) % 3, nit % 3, (nit + 1) % 3):
            wait_scatter(s)
        plsc.subcore_barrier()

        def drain(r0):
            pltpu.sync_copy(acc_sh.at[pl.ds(r0, _CH), :],
                            out_hbm.at[cid, pl.ds(r0, _CH), :])

        _sweep_chunks(n, _CH, sid, drain)

    return k(xs, src1d, dst1d)


def _tc_prep(degp, x, bn):
    n, d = x.shape

    def body(degp_ref, x_ref, dinv_ref, xs_ref):
        deg = degp_ref[0][:, 0:1] + degp_ref[1][:, 0:1] + 1.0
        dinv = lax.rsqrt(jnp.maximum(deg, 1.0))
        dinv_ref[...] = dinv
        xs_ref[...] = x_ref[...] * dinv

    return pl.pallas_call(
        body,
        grid=(n // bn,),
        in_specs=[
            pl.BlockSpec((_NC, bn, 16), lambda i: (0, i, 0)),
            pl.BlockSpec((bn, d), lambda i: (i, 0)),
        ],
        out_specs=[
            pl.BlockSpec((bn, 1), lambda i: (i, 0)),
            pl.BlockSpec((bn, d), lambda i: (i, 0)),
        ],
        out_shape=[
            jax.ShapeDtypeStruct((n, 1), jnp.float32),
            jax.ShapeDtypeStruct((n, d), jnp.float32),
        ],
    )(degp, x)


def _tc_layer(sp, xin, dinv, w, b, g, be, h1prev, bn):
    n, d = xin.shape
    nb = n // bn
    mid = h1prev is None

    def body(*refs):
        if mid:
            (sp_ref, x_ref, dinv_ref, w_ref, b_ref, g_ref, be_ref,
             o1_ref, o2_ref, hs, s1a, s2a) = refs
        else:
            (sp_ref, x_ref, dinv_ref, w_ref, b_ref, g_ref, be_ref, hp_ref,
             o1_ref, hs, s1a, s2a) = refs
        i = pl.program_id(0)

        @pl.when(i < nb)
        def _():
            dv = dinv_ref[...]
            agg = dv * (sp_ref[0] + sp_ref[1]) + (dv * dv) * x_ref[...]
            h = jnp.dot(agg, w_ref[...], preferred_element_type=jnp.float32)
            h = h + b_ref[...]
            hs[pl.ds(i * bn, bn), :] = h
            s1 = jnp.sum(h, axis=0, keepdims=True)
            s2 = jnp.sum(h * h, axis=0, keepdims=True)

            @pl.when(i == 0)
            def _():
                s1a[...] = s1
                s2a[...] = s2

            @pl.when(i > 0)
            def _():
                s1a[...] += s1
                s2a[...] += s2

        @pl.when(i >= nb)
        def _():
            mu = s1a[...] * (1.0 / n)
            ex2 = s2a[...] * (1.0 / n)
            rstd = lax.rsqrt(jnp.maximum(ex2 - mu * mu, 0.0) + 1e-5)
            hblk = hs[pl.ds((i - nb) * bn, bn), :]
            h1 = jnp.maximum((hblk - mu) * rstd * g_ref[...] + be_ref[...],
                             0.0)
            if mid:
                o1_ref[...] = h1
                o2_ref[...] = h1 * dinv_ref[...]
            else:
                o1_ref[...] = hp_ref[...] + h1

    lo = lambda i: (jnp.where(i < nb, i, 0), 0)
    hi = lambda i: (jnp.where(i < nb, 0, i - nb), 0)
    in_specs = [
        pl.BlockSpec((_NC, bn, d), lambda i: (0, jnp.where(i < nb, i, 0), 0)),
        pl.BlockSpec((bn, d), lo),
        pl.BlockSpec((bn, 1), lambda i: (i % nb, 0) if mid else lo(i)),
        pl.BlockSpec((d, d), lambda i: (0, 0)),
        pl.BlockSpec((1, d), lambda i: (0, 0)),
        pl.BlockSpec((1, d), lambda i: (0, 0)),
        pl.BlockSpec((1, d), lambda i: (0, 0)),
    ]
    args = [sp, xin, dinv, w, b, g, be]
    if mid:
        out_specs = [pl.BlockSpec((bn, d), hi), pl.BlockSpec((bn, d), hi)]
        out_shape = [jax.ShapeDtypeStruct((n, d), jnp.float32)] * 2
    else:
        in_specs.append(pl.BlockSpec((bn, d), hi))
        args.append(h1prev)
        out_specs = pl.BlockSpec((bn, d), hi)
        out_shape = jax.ShapeDtypeStruct((n, d), jnp.float32)

    return pl.pallas_call(
        body,
        grid=(2 * nb,),
        in_specs=in_specs,
        out_specs=out_specs,
        out_shape=out_shape,
        scratch_shapes=[
            pltpu.VMEM((n, d), jnp.float32),
            pltpu.VMEM((1, d), jnp.float32),
            pltpu.VMEM((1, d), jnp.float32),
        ],
    )(*args)


def kernel(x, edge_index, W1, b1, g1, be1, W2, b2, g2, be2):
    n, d = x.shape
    e = edge_index.shape[1]
    assert e % (_NW * _CH) == 0 and n % 200 == 0 and n % _CH == 0
    src1d = edge_index[0]
    dst1d = edge_index[1]
    b1r, g1r, be1r = b1.reshape(1, d), g1.reshape(1, d), be1.reshape(1, d)
    b2r, g2r, be2r = b2.reshape(1, d), g2.reshape(1, d), be2.reshape(1, d)
    bn = 1000

    degp = _sc_deg(dst1d, n)
    dinv, xs1 = _tc_prep(degp, x, bn)
    s1p = _sc_agg(xs1, src1d, dst1d)
    h1, xs2 = _tc_layer(s1p, x, dinv, W1, b1r, g1r, be1r, None, bn)
    s2p = _sc_agg(xs2, src1d, dst1d)
    return _tc_layer(s2p, h1, dinv, W2, b2r, g2r, be2r, h1, bn)
